# baseline (speedup 1.0000x reference)
import functools

import jax
import jax.numpy as jnp
from jax.experimental import pallas as pl
from jax.experimental.pallas import tpu as pltpu

NLAT_IN, NLON_IN = 361, 720
NLAT_OUT, NLON_OUT = 721, 1440


_RB = 8
_NR = 46


def _lon_lerp(v):
    vs = jnp.concatenate([v[:, 1:], v[:, :1]], axis=1)
    vo = v + 0.5 * (vs - v)
    return jnp.stack([v, vo], axis=-1).reshape(v.shape[0], NLON_OUT)


def _tc_body(a_ref, b_ref, o_ref):
    a = a_ref[0, 0]
    b = b_ref[0, 0]
    an = jnp.concatenate([a[1:], b[:1]], axis=0)
    e = _lon_lerp(a)
    o = e + 0.5 * (_lon_lerp(an) - e)
    inter = jnp.concatenate([e[:, None, :], o[:, None, :]], axis=1)
    o_ref[0, 0] = inter.reshape(2 * _RB, NLON_OUT)


def kernel(x):
    nb, nc = x.shape[0], x.shape[1]
    return pl.pallas_call(
        _tc_body,
        grid=(nc, _NR),
        in_specs=[
            pl.BlockSpec((1, 1, _RB, NLON_IN), lambda c, r: (0, c, r, 0)),
            pl.BlockSpec(
                (1, 1, _RB, NLON_IN), lambda c, r: (0, c, jnp.minimum(r + 1, _NR - 1), 0)
            ),
        ],
        out_specs=pl.BlockSpec((1, 1, 2 * _RB, NLON_OUT), lambda c, r: (0, c, r, 0)),
        out_shape=jax.ShapeDtypeStruct((nb, nc, NLAT_OUT, NLON_OUT), x.dtype),
    )(x, x)


# speedup vs baseline: 5.7223x; 5.7223x over previous
import functools

import jax
import jax.numpy as jnp
import numpy as np
from jax.experimental import pallas as pl
from jax.experimental.pallas import tpu as pltpu

NLAT_IN, NLON_IN = 361, 720
NLAT_OUT, NLON_OUT = 721, 1440


_RB = 8
_NR = 46


def _make_mats():
    lmat = np.zeros((16, 16), np.float32)
    for t in range(8):
        lmat[2 * t, t] = 1.0
        lmat[2 * t + 1, t] = 0.5
        lmat[2 * t + 1, t + 1] = 0.5
    wmat = np.zeros((NLON_IN, NLON_OUT), np.float32)
    for j in range(NLON_IN):
        wmat[j, 2 * j] = 1.0
        wmat[j, 2 * j + 1] += 0.5
        wmat[(j + 1) % NLON_IN, 2 * j + 1] += 0.5
    return jnp.asarray(lmat), jnp.asarray(wmat)


_LMAT, _WMAT = _make_mats()


def _tc_body(a_ref, b_ref, l_ref, w_ref, o_ref):
    a = a_ref[0, 0]
    b = b_ref[0, 0]
    af = jnp.concatenate([a, b], axis=0)
    r = pl.program_id(1)
    row_ids = _RB * r + jax.lax.broadcasted_iota(jnp.int32, (16, 1), 0)
    af = jnp.where(row_ids <= NLAT_IN - 1, af, 0.0)
    z = jax.lax.dot(
        l_ref[...], af, precision=jax.lax.Precision.HIGHEST
    )
    out = jax.lax.dot(
        z, w_ref[...], precision=jax.lax.Precision.HIGHEST
    )
    o_ref[0, 0] = out


def kernel(x):
    nb, nc = x.shape[0], x.shape[1]
    return pl.pallas_call(
        _tc_body,
        grid=(nc, _NR),
        in_specs=[
            pl.BlockSpec((1, 1, _RB, NLON_IN), lambda c, r: (0, c, r, 0)),
            pl.BlockSpec(
                (1, 1, _RB, NLON_IN), lambda c, r: (0, c, jnp.minimum(r + 1, _NR - 1), 0)
            ),
            pl.BlockSpec((16, 16), lambda c, r: (0, 0)),
            pl.BlockSpec((NLON_IN, NLON_OUT), lambda c, r: (0, 0)),
        ],
        out_specs=pl.BlockSpec((1, 1, 2 * _RB, NLON_OUT), lambda c, r: (0, c, r, 0)),
        out_shape=jax.ShapeDtypeStruct((nb, nc, NLAT_OUT, NLON_OUT), x.dtype),
    )(x, x, _LMAT, _WMAT)


# speedup vs baseline: 47.4354x; 8.2896x over previous
import functools

import jax
import jax.numpy as jnp
import numpy as np
from jax.experimental import pallas as pl
from jax.experimental.pallas import tpu as pltpu

NLAT_IN, NLON_IN = 361, 720
NLAT_OUT, NLON_OUT = 721, 1440


_RB = 64
_NR = 6


def _make_mats():
    lmat = np.zeros((2 * _RB, 2 * _RB), np.float32)
    for t in range(_RB):
        lmat[2 * t, t] = 1.0
        lmat[2 * t + 1, t] = 0.5
        lmat[2 * t + 1, t + 1] = 0.5
    wmat = np.zeros((NLON_IN, NLON_OUT), np.float32)
    for j in range(NLON_IN):
        wmat[j, 2 * j] = 1.0
        wmat[j, 2 * j + 1] += 0.5
        wmat[(j + 1) % NLON_IN, 2 * j + 1] += 0.5
    return jnp.asarray(lmat, jnp.bfloat16), jnp.asarray(wmat, jnp.bfloat16)


_LMAT, _WMAT = _make_mats()


def _split(m):
    hi = m.astype(jnp.bfloat16)
    lo = (m - hi.astype(jnp.float32)).astype(jnp.bfloat16)
    return hi, lo


def _dot2(ah, al, b):
    f32 = jnp.float32
    return jax.lax.dot(ah, b, preferred_element_type=f32) + jax.lax.dot(
        al, b, preferred_element_type=f32
    )


def _tc_body(a_ref, b_ref, l_ref, w_ref, o_ref):
    a = a_ref[0, 0]
    b = b_ref[0, 0]
    af = jnp.concatenate([a, b], axis=0)
    r = pl.program_id(1)
    row_ids = _RB * r + jax.lax.broadcasted_iota(jnp.int32, (2 * _RB, 1), 0)
    af = jnp.where(row_ids <= NLAT_IN - 1, af, 0.0)
    ah, al = _split(af)
    lm = l_ref[...]
    ylat = jax.lax.dot(lm, ah, preferred_element_type=jnp.float32) + jax.lax.dot(
        lm, al, preferred_element_type=jnp.float32
    )
    yh, yl = _split(ylat)
    o_ref[0, 0] = _dot2(yh, yl, w_ref[...])


def kernel(x):
    nb, nc = x.shape[0], x.shape[1]
    return pl.pallas_call(
        _tc_body,
        grid=(nc, _NR),
        in_specs=[
            pl.BlockSpec((1, 1, _RB, NLON_IN), lambda c, r: (0, c, r, 0)),
            pl.BlockSpec(
                (1, 1, _RB, NLON_IN), lambda c, r: (0, c, jnp.minimum(r + 1, _NR - 1), 0)
            ),
            pl.BlockSpec((2 * _RB, 2 * _RB), lambda c, r: (0, 0)),
            pl.BlockSpec((NLON_IN, NLON_OUT), lambda c, r: (0, 0)),
        ],
        out_specs=pl.BlockSpec((1, 1, 2 * _RB, NLON_OUT), lambda c, r: (0, c, r, 0)),
        out_shape=jax.ShapeDtypeStruct((nb, nc, NLAT_OUT, NLON_OUT), x.dtype),
    )(x, x, _LMAT, _WMAT)
